# baseline (device time: 198224 ns/iter reference)
import functools

import jax
import jax.numpy as jnp
from jax import lax
from jax.experimental import pallas as pl
from jax.experimental.pallas import tpu as pltpu

B, S, H, Dh, Dr = 4, 256, 32, 128, 64
D = 4096
DC_HALF = 128
M = B * S
F32 = jnp.float32
_VMEM_LIMIT = 60 * 1024 * 1024


Q_TILE = 256
QR_TILE = 256
N_Q = D // Q_TILE
N_STEPS = N_Q + 2048 // QR_TILE


def _exchange_qqr(x2d, Wdkv, Wuk, Wuv, Wkr, Wq, Wqr):

    def body(x_ref, wdkv_ref, wuk_ref, wuv_ref, wkr_ref, wq_ref,
             wqr_ref, c_ref, wukc_ref, wuvc_ref, kr_ref, q_ref, qr_ref,
             send_sems, recv_sems):
        j = pl.program_id(0)
        my_x = lax.axis_index("x")
        my_y = lax.axis_index("y")
        my_z = lax.axis_index("z")
        peer = (my_x, 1 - my_y, my_z)

        def mk(src, dst, i):
            return pltpu.make_async_remote_copy(
                src_ref=src, dst_ref=dst,
                send_sem=send_sems.at[i], recv_sem=recv_sems.at[i],
                device_id=peer, device_id_type=pl.DeviceIdType.MESH)

        @pl.when(j == 0)
        def _():
            barrier_sem = pltpu.get_barrier_semaphore()
            pl.semaphore_signal(barrier_sem, inc=1, device_id=peer,
                                device_id_type=pl.DeviceIdType.MESH)
            pl.semaphore_wait(barrier_sem, 1)
            mk(wuk_ref, wukc_ref.at[1], 0).start()
            mk(wuv_ref, wuvc_ref.at[1], 1).start()
            c_ref[0] = jnp.dot(x_ref[...], wdkv_ref[...],
                               preferred_element_type=F32)
            mk(c_ref.at[0], c_ref.at[1], 2).start()
            wukc_ref[0] = wuk_ref[...]
            wuvc_ref[0] = wuv_ref[...]
            kr_ref[...] = jnp.dot(x_ref[...], wkr_ref[...],
                                  preferred_element_type=F32)

        @pl.when(j < N_Q)
        def _():
            q_ref[...] = jnp.dot(x_ref[...], wq_ref[...],
                                 preferred_element_type=F32)

        @pl.when(j >= N_Q)
        def _():
            qr_ref[...] = jnp.dot(x_ref[...], wqr_ref[...],
                                  preferred_element_type=F32)

        @pl.when(j == N_STEPS - 1)
        def _():
            mk(wuk_ref, wukc_ref.at[1], 0).wait()
            mk(wuv_ref, wuvc_ref.at[1], 1).wait()
            mk(c_ref.at[0], c_ref.at[1], 2).wait()

    const2 = lambda j: (0, 0)
    const3 = lambda j: (0, 0, 0)
    return pl.pallas_call(
        body,
        grid=(N_STEPS,),
        in_specs=[
            pl.BlockSpec((M, D), const2),
            pl.BlockSpec((D, DC_HALF), const2),
            pl.BlockSpec((DC_HALF, D), const2),
            pl.BlockSpec((DC_HALF, D), const2),
            pl.BlockSpec((D, Dr), const2),
            pl.BlockSpec((D, Q_TILE),
                         lambda j: (0, jnp.minimum(j, N_Q - 1))),
            pl.BlockSpec((D, QR_TILE),
                         lambda j: (0, jnp.maximum(j - N_Q, 0))),
        ],
        out_specs=[
            pl.BlockSpec((2, M, DC_HALF), const3),
            pl.BlockSpec((2, DC_HALF, D), const3),
            pl.BlockSpec((2, DC_HALF, D), const3),
            pl.BlockSpec((M, Dr), const2),
            pl.BlockSpec((M, Q_TILE),
                         lambda j: (0, jnp.minimum(j, N_Q - 1))),
            pl.BlockSpec((M, QR_TILE),
                         lambda j: (0, jnp.maximum(j - N_Q, 0))),
        ],
        out_shape=[
            jax.ShapeDtypeStruct((2, M, DC_HALF), F32),
            jax.ShapeDtypeStruct((2, DC_HALF, D), F32),
            jax.ShapeDtypeStruct((2, DC_HALF, D), F32),
            jax.ShapeDtypeStruct((M, Dr), F32),
            jax.ShapeDtypeStruct((M, D), F32),
            jax.ShapeDtypeStruct((M, 2048), F32),
        ],
        scratch_shapes=[
            pltpu.SemaphoreType.DMA((3,)),
            pltpu.SemaphoreType.DMA((3,)),
        ],
        compiler_params=pltpu.CompilerParams(
            collective_id=0, vmem_limit_bytes=62 * 1024 * 1024),
    )(x2d, Wdkv, Wuk, Wuv, Wkr, Wq, Wqr)


def _matmul_nt(a, b, n_tile, cast_bf16=False):
    m, k = a.shape
    _, n = b.shape

    def body(a_ref, b_ref, o_ref):
        if cast_bf16:
            lhs = a_ref[...].astype(jnp.bfloat16)
            rhs = b_ref[...].astype(jnp.bfloat16)
        else:
            lhs = a_ref[...]
            rhs = b_ref[...]
        o_ref[...] = jnp.dot(lhs, rhs, preferred_element_type=F32)

    return pl.pallas_call(
        body,
        grid=(n // n_tile,),
        in_specs=[
            pl.BlockSpec((m, k), lambda j: (0, 0)),
            pl.BlockSpec((k, n_tile), lambda j: (0, j)),
        ],
        out_specs=pl.BlockSpec((m, n_tile), lambda j: (0, j)),
        out_shape=jax.ShapeDtypeStruct((m, n), F32),
        compiler_params=pltpu.CompilerParams(vmem_limit_bytes=_VMEM_LIMIT),
    )(a, b)


def _attn_fused(q3, qr3, kr3, c_cat, wuk_cat, wuv_cat):
    scale = (Dh + Dr) ** -0.5
    contract_last = (((1,), (1,)), ((), ()))

    def body(q_ref, qr_ref, kr_ref, c_ref, wuk_ref, wuv_ref, o_ref,
             k_s, v_s):
        c0 = c_ref[0]
        c1 = c_ref[1]
        k_s[...] = (jnp.dot(c0, wuk_ref[0], preferred_element_type=F32)
                    + jnp.dot(c1, wuk_ref[1], preferred_element_type=F32))
        v_s[...] = (jnp.dot(c0, wuv_ref[0], preferred_element_type=F32)
                    + jnp.dot(c1, wuv_ref[1], preferred_element_type=F32))
        kr = kr_ref[0, :, :]
        for h in range(H):
            q = q_ref[0, :, h * Dh:(h + 1) * Dh]
            k = k_s[:, h * Dh:(h + 1) * Dh]
            v = v_s[:, h * Dh:(h + 1) * Dh]
            qr = qr_ref[0, :, h * Dr:(h + 1) * Dr]
            s = (lax.dot_general(q, k, contract_last,
                                 preferred_element_type=F32)
                 + lax.dot_general(qr, kr, contract_last,
                                   preferred_element_type=F32)) * scale
            p = jnp.exp(s)
            r = 1.0 / jnp.sum(p, axis=-1, keepdims=True)
            o_ref[0, :, h * Dh:(h + 1) * Dh] = jnp.dot(
                p, v, preferred_element_type=F32) * r

    return pl.pallas_call(
        body,
        grid=(B,),
        in_specs=[
            pl.BlockSpec((1, S, H * Dh), lambda b: (b, 0, 0)),
            pl.BlockSpec((1, S, H * Dr), lambda b: (b, 0, 0)),
            pl.BlockSpec((1, S, Dr), lambda b: (b, 0, 0)),
            pl.BlockSpec((2, S, DC_HALF), lambda b: (0, b, 0)),
            pl.BlockSpec((2, DC_HALF, D), lambda b: (0, 0, 0)),
            pl.BlockSpec((2, DC_HALF, D), lambda b: (0, 0, 0)),
        ],
        out_specs=pl.BlockSpec((1, S, H * Dh), lambda b: (b, 0, 0)),
        out_shape=jax.ShapeDtypeStruct((B, S, H * Dh), F32),
        scratch_shapes=[
            pltpu.VMEM((S, H * Dh), F32),
            pltpu.VMEM((S, H * Dh), F32),
        ],
        compiler_params=pltpu.CompilerParams(vmem_limit_bytes=_VMEM_LIMIT),
    )(q3, qr3, kr3, c_cat, wuk_cat, wuv_cat)


def kernel(x, Wdkv, Wuk, Wuv, Wq, Wqr, Wkr, Wo):
    x2d = x.reshape(M, D)
    c_cat, wuk_cat, wuv_cat, kr, q, qr = _exchange_qqr(
        x2d, Wdkv, Wuk, Wuv, Wkr, Wq, Wqr)
    o = _attn_fused(
        q.reshape(B, S, H * Dh),
        qr.reshape(B, S, H * Dr),
        kr.reshape(B, S, Dr),
        c_cat, wuk_cat, wuv_cat,
    )
    out = _matmul_nt(o.reshape(M, H * Dh), Wo, 256, cast_bf16=True)
    return out.reshape(B, S, D)


# device time: 181657 ns/iter; 1.0912x vs baseline; 1.0912x over previous
import functools

import jax
import jax.numpy as jnp
from jax import lax
from jax.experimental import pallas as pl
from jax.experimental.pallas import tpu as pltpu

B, S, H, Dh, Dr = 4, 256, 32, 128, 64
D = 4096
DC_HALF = 128
M = B * S
F32 = jnp.float32
_VMEM_LIMIT = 60 * 1024 * 1024


Q_TILE = 256
QR_TILE = 256
N_Q = D // Q_TILE
N_STEPS = N_Q + 2048 // QR_TILE


def _exchange_qqr(x2d, Wdkv, Wuk, Wuv, Wkr, Wq, Wqr):

    def body(x_ref, wdkv_ref, wuk_ref, wuv_ref, wkr_ref, wq_ref,
             wqr_ref, c_ref, wukc_ref, wuvc_ref, kr_ref, q_ref, qr_ref,
             send_sems, recv_sems):
        j = pl.program_id(0)
        my_x = lax.axis_index("x")
        my_y = lax.axis_index("y")
        my_z = lax.axis_index("z")
        peer = (my_x, 1 - my_y, my_z)

        def mk(src, dst, i):
            return pltpu.make_async_remote_copy(
                src_ref=src, dst_ref=dst,
                send_sem=send_sems.at[i], recv_sem=recv_sems.at[i],
                device_id=peer, device_id_type=pl.DeviceIdType.MESH)

        @pl.when(j == 0)
        def _():
            barrier_sem = pltpu.get_barrier_semaphore()
            pl.semaphore_signal(barrier_sem, inc=1, device_id=peer,
                                device_id_type=pl.DeviceIdType.MESH)
            pl.semaphore_wait(barrier_sem, 1)
            mk(wuk_ref, wukc_ref.at[1], 0).start()
            mk(wuv_ref, wuvc_ref.at[1], 1).start()
            c_ref[0] = jnp.dot(x_ref[...], wdkv_ref[...],
                               preferred_element_type=F32)
            mk(c_ref.at[0], c_ref.at[1], 2).start()
            wukc_ref[0] = wuk_ref[...]
            wuvc_ref[0] = wuv_ref[...]
            kr_ref[...] = jnp.dot(x_ref[...], wkr_ref[...],
                                  preferred_element_type=F32)

        @pl.when(j < N_Q)
        def _():
            q_ref[...] = jnp.dot(x_ref[...], wq_ref[...],
                                 preferred_element_type=F32)

        @pl.when(j >= N_Q)
        def _():
            qr_ref[...] = jnp.dot(x_ref[...], wqr_ref[...],
                                  preferred_element_type=F32)

        @pl.when(j == N_STEPS - 1)
        def _():
            mk(wuk_ref, wukc_ref.at[1], 0).wait()
            mk(wuv_ref, wuvc_ref.at[1], 1).wait()
            mk(c_ref.at[0], c_ref.at[1], 2).wait()

    const2 = lambda j: (0, 0)
    const3 = lambda j: (0, 0, 0)
    return pl.pallas_call(
        body,
        grid=(N_STEPS,),
        in_specs=[
            pl.BlockSpec((M, D), const2),
            pl.BlockSpec((D, DC_HALF), const2),
            pl.BlockSpec((DC_HALF, D), const2),
            pl.BlockSpec((DC_HALF, D), const2),
            pl.BlockSpec((D, Dr), const2),
            pl.BlockSpec((D, Q_TILE),
                         lambda j: (0, jnp.minimum(j, N_Q - 1))),
            pl.BlockSpec((D, QR_TILE),
                         lambda j: (0, jnp.maximum(j - N_Q, 0))),
        ],
        out_specs=[
            pl.BlockSpec((2, M, DC_HALF), const3),
            pl.BlockSpec((2, DC_HALF, D), const3),
            pl.BlockSpec((2, DC_HALF, D), const3),
            pl.BlockSpec((M, Dr), const2),
            pl.BlockSpec((M, Q_TILE),
                         lambda j: (0, jnp.minimum(j, N_Q - 1))),
            pl.BlockSpec((M, QR_TILE),
                         lambda j: (0, jnp.maximum(j - N_Q, 0))),
        ],
        out_shape=[
            jax.ShapeDtypeStruct((2, M, DC_HALF), F32),
            jax.ShapeDtypeStruct((2, DC_HALF, D), F32),
            jax.ShapeDtypeStruct((2, DC_HALF, D), F32),
            jax.ShapeDtypeStruct((M, Dr), F32),
            jax.ShapeDtypeStruct((M, D), F32),
            jax.ShapeDtypeStruct((M, 2048), F32),
        ],
        scratch_shapes=[
            pltpu.SemaphoreType.DMA((3,)),
            pltpu.SemaphoreType.DMA((3,)),
        ],
        compiler_params=pltpu.CompilerParams(
            collective_id=0, vmem_limit_bytes=62 * 1024 * 1024),
    )(x2d, Wdkv, Wuk, Wuv, Wkr, Wq, Wqr)


def _matmul_nt(a, b, n_tile, cast_bf16=False):
    m, k = a.shape
    _, n = b.shape

    def body(a_ref, b_ref, o_ref):
        if cast_bf16:
            lhs = a_ref[...].astype(jnp.bfloat16)
            rhs = b_ref[...].astype(jnp.bfloat16)
        else:
            lhs = a_ref[...]
            rhs = b_ref[...]
        o_ref[...] = jnp.dot(lhs, rhs, preferred_element_type=F32)

    return pl.pallas_call(
        body,
        grid=(n // n_tile,),
        in_specs=[
            pl.BlockSpec((m, k), lambda j: (0, 0)),
            pl.BlockSpec((k, n_tile), lambda j: (0, j)),
        ],
        out_specs=pl.BlockSpec((m, n_tile), lambda j: (0, j)),
        out_shape=jax.ShapeDtypeStruct((m, n), F32),
        compiler_params=pltpu.CompilerParams(vmem_limit_bytes=_VMEM_LIMIT),
    )(a, b)


def _attn_fused(q3, qr3, kr3, c_cat, wuk_cat, wuv_cat):
    scale = (Dh + Dr) ** -0.5
    contract_last = (((1,), (1,)), ((), ()))

    def body(q_ref, qr_ref, kr_ref, c_ref, wuk_ref, wuv_ref, o_ref,
             k_s, v_s):
        c0 = c_ref[0]
        c1 = c_ref[1]
        k_s[...] = (jnp.dot(c0, wuk_ref[0], preferred_element_type=F32)
                    + jnp.dot(c1, wuk_ref[1], preferred_element_type=F32))
        v_s[...] = (jnp.dot(c0, wuv_ref[0], preferred_element_type=F32)
                    + jnp.dot(c1, wuv_ref[1], preferred_element_type=F32))
        kr = kr_ref[0, :, :]
        ones = jnp.ones((S, 128), F32)
        for h in range(H):
            q = q_ref[0, :, h * Dh:(h + 1) * Dh]
            k = k_s[:, h * Dh:(h + 1) * Dh]
            v = v_s[:, h * Dh:(h + 1) * Dh]
            qr = qr_ref[0, :, h * Dr:(h + 1) * Dr]
            s = (lax.dot_general(q, k, contract_last,
                                 preferred_element_type=F32)
                 + lax.dot_general(qr, kr, contract_last,
                                   preferred_element_type=F32)) * scale
            p = jnp.exp(s)
            l = jnp.dot(p, ones, preferred_element_type=F32)
            r = 1.0 / l[:, 0:1]
            o_ref[0, :, h * Dh:(h + 1) * Dh] = jnp.dot(
                p, v, preferred_element_type=F32) * r

    return pl.pallas_call(
        body,
        grid=(B,),
        in_specs=[
            pl.BlockSpec((1, S, H * Dh), lambda b: (b, 0, 0)),
            pl.BlockSpec((1, S, H * Dr), lambda b: (b, 0, 0)),
            pl.BlockSpec((1, S, Dr), lambda b: (b, 0, 0)),
            pl.BlockSpec((2, S, DC_HALF), lambda b: (0, b, 0)),
            pl.BlockSpec((2, DC_HALF, D), lambda b: (0, 0, 0)),
            pl.BlockSpec((2, DC_HALF, D), lambda b: (0, 0, 0)),
        ],
        out_specs=pl.BlockSpec((1, S, H * Dh), lambda b: (b, 0, 0)),
        out_shape=jax.ShapeDtypeStruct((B, S, H * Dh), F32),
        scratch_shapes=[
            pltpu.VMEM((S, H * Dh), F32),
            pltpu.VMEM((S, H * Dh), F32),
        ],
        compiler_params=pltpu.CompilerParams(vmem_limit_bytes=_VMEM_LIMIT),
    )(q3, qr3, kr3, c_cat, wuk_cat, wuv_cat)


def kernel(x, Wdkv, Wuk, Wuv, Wq, Wqr, Wkr, Wo):
    x2d = x.reshape(M, D)
    c_cat, wuk_cat, wuv_cat, kr, q, qr = _exchange_qqr(
        x2d, Wdkv, Wuk, Wuv, Wkr, Wq, Wqr)
    o = _attn_fused(
        q.reshape(B, S, H * Dh),
        qr.reshape(B, S, H * Dr),
        kr.reshape(B, S, Dr),
        c_cat, wuk_cat, wuv_cat,
    )
    out = _matmul_nt(o.reshape(M, H * Dh), Wo, 512, cast_bf16=True)
    return out.reshape(B, S, D)


# device time: 176145 ns/iter; 1.1253x vs baseline; 1.0313x over previous
import functools

import jax
import jax.numpy as jnp
from jax import lax
from jax.experimental import pallas as pl
from jax.experimental.pallas import tpu as pltpu

B, S, H, Dh, Dr = 4, 256, 32, 128, 64
D = 4096
DC_HALF = 128
M = B * S
F32 = jnp.float32
_VMEM_LIMIT = 60 * 1024 * 1024


Q_TILE = 256
QR_TILE = 256
N_Q = D // Q_TILE
N_STEPS = N_Q + 2048 // QR_TILE


def _exchange_qqr(x2d, Wdkv, Wuk, Wuv, Wkr, Wq, Wqr):

    def body(x_ref, wdkv_ref, wuk_ref, wuv_ref, wkr_ref, wq_ref,
             wqr_ref, c_ref, wukc_ref, wuvc_ref, kr_ref, q_ref, qr_ref,
             send_sems, recv_sems):
        j = pl.program_id(0)
        my_x = lax.axis_index("x")
        my_y = lax.axis_index("y")
        my_z = lax.axis_index("z")
        peer = (my_x, 1 - my_y, my_z)

        def mk(src, dst, i):
            return pltpu.make_async_remote_copy(
                src_ref=src, dst_ref=dst,
                send_sem=send_sems.at[i], recv_sem=recv_sems.at[i],
                device_id=peer, device_id_type=pl.DeviceIdType.MESH)

        @pl.when(j == 0)
        def _():
            barrier_sem = pltpu.get_barrier_semaphore()
            pl.semaphore_signal(barrier_sem, inc=1, device_id=peer,
                                device_id_type=pl.DeviceIdType.MESH)
            pl.semaphore_wait(barrier_sem, 1)
            mk(wuk_ref, wukc_ref.at[1], 0).start()
            mk(wuv_ref, wuvc_ref.at[1], 1).start()
            c_ref[0] = jnp.dot(x_ref[...], wdkv_ref[...],
                               preferred_element_type=F32)
            mk(c_ref.at[0], c_ref.at[1], 2).start()
            wukc_ref[0] = wuk_ref[...]
            wuvc_ref[0] = wuv_ref[...]
            kr_ref[...] = jnp.dot(x_ref[...], wkr_ref[...],
                                  preferred_element_type=F32)

        @pl.when(j < N_Q)
        def _():
            q_ref[...] = jnp.dot(x_ref[...], wq_ref[...],
                                 preferred_element_type=F32)

        @pl.when(j >= N_Q)
        def _():
            qr_ref[...] = jnp.dot(x_ref[...], wqr_ref[...],
                                  preferred_element_type=F32)

        @pl.when(j == N_STEPS - 1)
        def _():
            mk(wuk_ref, wukc_ref.at[1], 0).wait()
            mk(wuv_ref, wuvc_ref.at[1], 1).wait()
            mk(c_ref.at[0], c_ref.at[1], 2).wait()

    const2 = lambda j: (0, 0)
    const3 = lambda j: (0, 0, 0)
    return pl.pallas_call(
        body,
        grid=(N_STEPS,),
        in_specs=[
            pl.BlockSpec((M, D), const2),
            pl.BlockSpec((D, DC_HALF), const2),
            pl.BlockSpec((DC_HALF, D), const2),
            pl.BlockSpec((DC_HALF, D), const2),
            pl.BlockSpec((D, Dr), const2),
            pl.BlockSpec((D, Q_TILE),
                         lambda j: (0, jnp.minimum(j, N_Q - 1))),
            pl.BlockSpec((D, QR_TILE),
                         lambda j: (0, jnp.maximum(j - N_Q, 0))),
        ],
        out_specs=[
            pl.BlockSpec((2, M, DC_HALF), const3),
            pl.BlockSpec((2, DC_HALF, D), const3),
            pl.BlockSpec((2, DC_HALF, D), const3),
            pl.BlockSpec((M, Dr), const2),
            pl.BlockSpec((M, Q_TILE),
                         lambda j: (0, jnp.minimum(j, N_Q - 1))),
            pl.BlockSpec((M, QR_TILE),
                         lambda j: (0, jnp.maximum(j - N_Q, 0))),
        ],
        out_shape=[
            jax.ShapeDtypeStruct((2, M, DC_HALF), F32),
            jax.ShapeDtypeStruct((2, DC_HALF, D), F32),
            jax.ShapeDtypeStruct((2, DC_HALF, D), F32),
            jax.ShapeDtypeStruct((M, Dr), F32),
            jax.ShapeDtypeStruct((M, D), F32),
            jax.ShapeDtypeStruct((M, 2048), F32),
        ],
        scratch_shapes=[
            pltpu.SemaphoreType.DMA((3,)),
            pltpu.SemaphoreType.DMA((3,)),
        ],
        compiler_params=pltpu.CompilerParams(
            collective_id=0, vmem_limit_bytes=62 * 1024 * 1024),
    )(x2d, Wdkv, Wuk, Wuv, Wkr, Wq, Wqr)


def _matmul_nt(a, b, n_tile, cast_bf16=False):
    m, k = a.shape
    _, n = b.shape

    def body(a_ref, b_ref, o_ref):
        if cast_bf16:
            lhs = a_ref[...].astype(jnp.bfloat16)
            rhs = b_ref[...].astype(jnp.bfloat16)
        else:
            lhs = a_ref[...]
            rhs = b_ref[...]
        o_ref[...] = jnp.dot(lhs, rhs, preferred_element_type=F32)

    return pl.pallas_call(
        body,
        grid=(n // n_tile,),
        in_specs=[
            pl.BlockSpec((m, k), lambda j: (0, 0)),
            pl.BlockSpec((k, n_tile), lambda j: (0, j)),
        ],
        out_specs=pl.BlockSpec((m, n_tile), lambda j: (0, j)),
        out_shape=jax.ShapeDtypeStruct((m, n), F32),
        compiler_params=pltpu.CompilerParams(vmem_limit_bytes=_VMEM_LIMIT),
    )(a, b)


WO_TILE = 256
N_WO = D // WO_TILE


def _attn_wo(q3, qr3, kr3, c_cat, wuk_cat, wuv_cat, Wo):
    scale = (Dh + Dr) ** -0.5
    contract_last = (((1,), (1,)), ((), ()))

    def body(q_ref, qr_ref, kr_ref, c_ref, wuk_ref, wuv_ref, wo_ref,
             out_ref, o_s, k_s, v_s):
        j = pl.program_id(0)

        @pl.when(j < B)
        def _():
            c0 = c_ref[0]
            c1 = c_ref[1]
            k_s[...] = (jnp.dot(c0, wuk_ref[0], preferred_element_type=F32)
                        + jnp.dot(c1, wuk_ref[1],
                                  preferred_element_type=F32))
            v_s[...] = (jnp.dot(c0, wuv_ref[0], preferred_element_type=F32)
                        + jnp.dot(c1, wuv_ref[1],
                                  preferred_element_type=F32))
            kr = kr_ref[0, :, :]
            ones = jnp.ones((S, 128), F32)
            ob_s = o_s.at[pl.ds(j * S, S), :]
            for h in range(H):
                q = q_ref[0, :, h * Dh:(h + 1) * Dh]
                k = k_s[:, h * Dh:(h + 1) * Dh]
                v = v_s[:, h * Dh:(h + 1) * Dh]
                qr = qr_ref[0, :, h * Dr:(h + 1) * Dr]
                s = (lax.dot_general(q, k, contract_last,
                                     preferred_element_type=F32)
                     + lax.dot_general(qr, kr, contract_last,
                                       preferred_element_type=F32)) * scale
                p = jnp.exp(s)
                l = jnp.dot(p, ones, preferred_element_type=F32)
                r = 1.0 / l[:, 0:1]
                ob_s[:, h * Dh:(h + 1) * Dh] = jnp.dot(
                    p, v, preferred_element_type=F32) * r

        @pl.when(j >= B)
        def _():
            out_ref[...] = jnp.dot(o_s[...], wo_ref[...],
                                   preferred_element_type=F32)

    batch_of = lambda j: jnp.minimum(j, B - 1)
    tile_of = lambda j: jnp.maximum(j - B, 0)
    return pl.pallas_call(
        body,
        grid=(B + N_WO,),
        in_specs=[
            pl.BlockSpec((1, S, H * Dh), lambda j: (batch_of(j), 0, 0)),
            pl.BlockSpec((1, S, H * Dr), lambda j: (batch_of(j), 0, 0)),
            pl.BlockSpec((1, S, Dr), lambda j: (batch_of(j), 0, 0)),
            pl.BlockSpec((2, S, DC_HALF), lambda j: (0, batch_of(j), 0)),
            pl.BlockSpec((2, DC_HALF, D), lambda j: (0, 0, 0)),
            pl.BlockSpec((2, DC_HALF, D), lambda j: (0, 0, 0)),
            pl.BlockSpec((H * Dh, WO_TILE), lambda j: (0, tile_of(j))),
        ],
        out_specs=pl.BlockSpec((M, WO_TILE), lambda j: (0, tile_of(j))),
        out_shape=jax.ShapeDtypeStruct((M, D), F32),
        scratch_shapes=[
            pltpu.VMEM((M, H * Dh), F32),
            pltpu.VMEM((S, H * Dh), F32),
            pltpu.VMEM((S, H * Dh), F32),
        ],
        compiler_params=pltpu.CompilerParams(vmem_limit_bytes=_VMEM_LIMIT),
    )(q3, qr3, kr3, c_cat, wuk_cat, wuv_cat, Wo)


def kernel(x, Wdkv, Wuk, Wuv, Wq, Wqr, Wkr, Wo):
    x2d = x.reshape(M, D)
    c_cat, wuk_cat, wuv_cat, kr, q, qr = _exchange_qqr(
        x2d, Wdkv, Wuk, Wuv, Wkr, Wq, Wqr)
    out = _attn_wo(
        q.reshape(B, S, H * Dh),
        qr.reshape(B, S, H * Dr),
        kr.reshape(B, S, Dr),
        c_cat, wuk_cat, wuv_cat, Wo,
    )
    return out.reshape(B, S, D)


# device time: 172951 ns/iter; 1.1461x vs baseline; 1.0185x over previous
import functools

import jax
import jax.numpy as jnp
from jax import lax
from jax.experimental import pallas as pl
from jax.experimental.pallas import tpu as pltpu

B, S, H, Dh, Dr = 4, 256, 32, 128, 64
D = 4096
DC_HALF = 128
M = B * S
F32 = jnp.float32
_VMEM_LIMIT = 60 * 1024 * 1024


Q_TILE = 512
QR_TILE = 256
N_Q = D // Q_TILE
N_STEPS = N_Q + 2048 // QR_TILE


def _exchange_qqr(x2d, Wdkv, Wuk, Wuv, Wkr, Wq, Wqr):

    def body(x_ref, wdkv_ref, wuk_ref, wuv_ref, wkr_ref, wq_ref,
             wqr_ref, c_ref, wukp_ref, wuvp_ref, kr_ref, q_ref, qr_ref,
             send_sems, recv_sems):
        j = pl.program_id(0)
        my_x = lax.axis_index("x")
        my_y = lax.axis_index("y")
        my_z = lax.axis_index("z")
        peer = (my_x, 1 - my_y, my_z)

        def mk(src, dst, i):
            return pltpu.make_async_remote_copy(
                src_ref=src, dst_ref=dst,
                send_sem=send_sems.at[i], recv_sem=recv_sems.at[i],
                device_id=peer, device_id_type=pl.DeviceIdType.MESH)

        @pl.when(j == 0)
        def _():
            barrier_sem = pltpu.get_barrier_semaphore()
            pl.semaphore_signal(barrier_sem, inc=1, device_id=peer,
                                device_id_type=pl.DeviceIdType.MESH)
            pl.semaphore_wait(barrier_sem, 1)
            mk(wuk_ref, wukp_ref, 0).start()
            mk(wuv_ref, wuvp_ref, 1).start()
            c_ref[0] = jnp.dot(x_ref[...], wdkv_ref[...],
                               preferred_element_type=F32)
            mk(c_ref.at[0], c_ref.at[1], 2).start()
            kr_ref[...] = jnp.dot(x_ref[...], wkr_ref[...],
                                  preferred_element_type=F32)

        @pl.when(j < N_Q)
        def _():
            q_ref[...] = jnp.dot(x_ref[...], wq_ref[...],
                                 preferred_element_type=F32)

        @pl.when(j >= N_Q)
        def _():
            qr_ref[...] = jnp.dot(x_ref[...], wqr_ref[...],
                                  preferred_element_type=F32)

        @pl.when(j == N_STEPS - 1)
        def _():
            mk(wuk_ref, wukp_ref, 0).wait()
            mk(wuv_ref, wuvp_ref, 1).wait()
            mk(c_ref.at[0], c_ref.at[1], 2).wait()

    const2 = lambda j: (0, 0)
    const3 = lambda j: (0, 0, 0)
    return pl.pallas_call(
        body,
        grid=(N_STEPS,),
        in_specs=[
            pl.BlockSpec((M, D), const2),
            pl.BlockSpec((D, DC_HALF), const2),
            pl.BlockSpec((DC_HALF, D), const2),
            pl.BlockSpec((DC_HALF, D), const2),
            pl.BlockSpec((D, Dr), const2),
            pl.BlockSpec((D, Q_TILE),
                         lambda j: (0, jnp.minimum(j, N_Q - 1))),
            pl.BlockSpec((D, QR_TILE),
                         lambda j: (0, jnp.maximum(j - N_Q, 0))),
        ],
        out_specs=[
            pl.BlockSpec((2, M, DC_HALF), const3),
            pl.BlockSpec((DC_HALF, D), const2),
            pl.BlockSpec((DC_HALF, D), const2),
            pl.BlockSpec((M, Dr), const2),
            pl.BlockSpec((M, Q_TILE),
                         lambda j: (0, jnp.minimum(j, N_Q - 1))),
            pl.BlockSpec((M, QR_TILE),
                         lambda j: (0, jnp.maximum(j - N_Q, 0))),
        ],
        out_shape=[
            jax.ShapeDtypeStruct((2, M, DC_HALF), F32),
            jax.ShapeDtypeStruct((DC_HALF, D), F32),
            jax.ShapeDtypeStruct((DC_HALF, D), F32),
            jax.ShapeDtypeStruct((M, Dr), F32),
            jax.ShapeDtypeStruct((M, D), F32),
            jax.ShapeDtypeStruct((M, 2048), F32),
        ],
        scratch_shapes=[
            pltpu.SemaphoreType.DMA((3,)),
            pltpu.SemaphoreType.DMA((3,)),
        ],
        compiler_params=pltpu.CompilerParams(
            collective_id=0, vmem_limit_bytes=62 * 1024 * 1024),
    )(x2d, Wdkv, Wuk, Wuv, Wkr, Wq, Wqr)


def _matmul_nt(a, b, n_tile, cast_bf16=False):
    m, k = a.shape
    _, n = b.shape

    def body(a_ref, b_ref, o_ref):
        if cast_bf16:
            lhs = a_ref[...].astype(jnp.bfloat16)
            rhs = b_ref[...].astype(jnp.bfloat16)
        else:
            lhs = a_ref[...]
            rhs = b_ref[...]
        o_ref[...] = jnp.dot(lhs, rhs, preferred_element_type=F32)

    return pl.pallas_call(
        body,
        grid=(n // n_tile,),
        in_specs=[
            pl.BlockSpec((m, k), lambda j: (0, 0)),
            pl.BlockSpec((k, n_tile), lambda j: (0, j)),
        ],
        out_specs=pl.BlockSpec((m, n_tile), lambda j: (0, j)),
        out_shape=jax.ShapeDtypeStruct((m, n), F32),
        compiler_params=pltpu.CompilerParams(vmem_limit_bytes=_VMEM_LIMIT),
    )(a, b)


WO_TILE = 256
N_WO = D // WO_TILE


def _attn_wo(q3, qr3, kr3, c_cat, wuk_mine, wuk_peer, wuv_mine,
             wuv_peer, Wo):
    scale = (Dh + Dr) ** -0.5
    contract_last = (((1,), (1,)), ((), ()))

    def body(q_ref, qr_ref, kr_ref, c_ref, wukm_ref, wukp_ref,
             wuvm_ref, wuvp_ref, wo_ref, out_ref, o_s, k_s, v_s):
        j = pl.program_id(0)

        @pl.when(j < B)
        def _():
            c0 = c_ref[0]
            c1 = c_ref[1]
            k_s[...] = (jnp.dot(c0, wukm_ref[...],
                                preferred_element_type=F32)
                        + jnp.dot(c1, wukp_ref[...],
                                  preferred_element_type=F32))
            v_s[...] = (jnp.dot(c0, wuvm_ref[...],
                                preferred_element_type=F32)
                        + jnp.dot(c1, wuvp_ref[...],
                                  preferred_element_type=F32))
            kr = kr_ref[0, :, :]
            ones = jnp.ones((S, 128), F32)
            ob_s = o_s.at[pl.ds(j * S, S), :]
            for h in range(H):
                q = q_ref[0, :, h * Dh:(h + 1) * Dh]
                k = k_s[:, h * Dh:(h + 1) * Dh]
                v = v_s[:, h * Dh:(h + 1) * Dh]
                qr = qr_ref[0, :, h * Dr:(h + 1) * Dr]
                s = (lax.dot_general(q, k, contract_last,
                                     preferred_element_type=F32)
                     + lax.dot_general(qr, kr, contract_last,
                                       preferred_element_type=F32)) * scale
                p = jnp.exp(s)
                l = jnp.dot(p, ones, preferred_element_type=F32)
                r = 1.0 / l[:, 0:1]
                ob_s[:, h * Dh:(h + 1) * Dh] = jnp.dot(
                    p, v, preferred_element_type=F32) * r

        @pl.when(j >= B)
        def _():
            out_ref[...] = jnp.dot(o_s[...], wo_ref[...],
                                   preferred_element_type=F32)

    batch_of = lambda j: jnp.minimum(j, B - 1)
    tile_of = lambda j: jnp.maximum(j - B, 0)
    return pl.pallas_call(
        body,
        grid=(B + N_WO,),
        in_specs=[
            pl.BlockSpec((1, S, H * Dh), lambda j: (batch_of(j), 0, 0)),
            pl.BlockSpec((1, S, H * Dr), lambda j: (batch_of(j), 0, 0)),
            pl.BlockSpec((1, S, Dr), lambda j: (batch_of(j), 0, 0)),
            pl.BlockSpec((2, S, DC_HALF), lambda j: (0, batch_of(j), 0)),
            pl.BlockSpec((DC_HALF, D), lambda j: (0, 0)),
            pl.BlockSpec((DC_HALF, D), lambda j: (0, 0)),
            pl.BlockSpec((DC_HALF, D), lambda j: (0, 0)),
            pl.BlockSpec((DC_HALF, D), lambda j: (0, 0)),
            pl.BlockSpec((H * Dh, WO_TILE), lambda j: (0, tile_of(j))),
        ],
        out_specs=pl.BlockSpec((M, WO_TILE), lambda j: (0, tile_of(j))),
        out_shape=jax.ShapeDtypeStruct((M, D), F32),
        scratch_shapes=[
            pltpu.VMEM((M, H * Dh), F32),
            pltpu.VMEM((S, H * Dh), F32),
            pltpu.VMEM((S, H * Dh), F32),
        ],
        compiler_params=pltpu.CompilerParams(vmem_limit_bytes=_VMEM_LIMIT),
    )(q3, qr3, kr3, c_cat, wuk_mine, wuk_peer, wuv_mine, wuv_peer, Wo)


def kernel(x, Wdkv, Wuk, Wuv, Wq, Wqr, Wkr, Wo):
    x2d = x.reshape(M, D)
    c_cat, wuk_peer, wuv_peer, kr, q, qr = _exchange_qqr(
        x2d, Wdkv, Wuk, Wuv, Wkr, Wq, Wqr)
    out = _attn_wo(
        q.reshape(B, S, H * Dh),
        qr.reshape(B, S, H * Dr),
        kr.reshape(B, S, Dr),
        c_cat, Wuk, wuk_peer, Wuv, wuv_peer, Wo,
    )
    return out.reshape(B, S, D)
